# baseline (speedup 1.0000x reference)
import functools

import jax
import jax.numpy as jnp
from jax import lax
from jax.experimental import pallas as pl
from jax.experimental.pallas import tpu as pltpu
from jax.experimental.pallas import tpu_sc as plsc

B = 16384
V = 1000000
D = 64
H1 = 384
H2 = 192
NC_OUT = 173
EPS = 1e-5

_NW = 32
_NT = (V + 127) // 128
_TPW = (_NT + _NW - 1) // _NW
_K = 2
_DUMP = B
_OUT_ROWS = B + 16


@functools.partial(
    pl.kernel,
    out_type=jax.ShapeDtypeStruct((_OUT_ROWS, 128), jnp.float32),
    mesh=plsc.VectorSubcoreMesh(core_axis_name="c", subcore_axis_name="s"),
    scratch_types=[
        pltpu.VMEM((B,), jnp.int32),
        pltpu.VMEM((B + 16,), jnp.int32),
        pltpu.VMEM((B + 16,), jnp.int32),
        pltpu.VMEM((2, _K, D, 128), jnp.float32),
        pltpu.VMEM((16, 128), jnp.float32),
        pltpu.VMEM((16,), jnp.int32),
        pltpu.SemaphoreType.DMA,
        pltpu.SemaphoreType.DMA,
        pltpu.SemaphoreType.DMA,
    ],
    compiler_params=pltpu.CompilerParams(
        use_tc_tiling_on_sc=True, needs_layout_passes=False,
        disable_bounds_checks=True),
)
def _sc_gather(table_hbm, idx_hbm, out_hbm, idx_v, pp_v, pc_v,
               stage_v, rows_v, pos_v, sem_a, sem_b, sem_o):
    wid = lax.axis_index("s") * 2 + lax.axis_index("c")
    t_start = wid * _TPW
    t_end = jnp.minimum(t_start + _TPW, _NT)
    pltpu.sync_copy(idx_hbm, idx_v)
    lane = lax.broadcasted_iota(jnp.int32, (16,), 0)

    def scan(i, off):
        r = idx_v[pl.ds(i * 16, 16)]
        t = lax.shift_right_logical(r, 7)
        m = (t >= t_start) & (t < t_end)
        p = lane + i * 16
        plsc.store_compressed(pp_v.at[pl.ds(off, 16)], p, mask=m)
        return off + plsc.all_reduce_population_count(m)[0]

    nloc = lax.fori_loop(0, B // 16, scan, 0)
    nloc_g = (nloc + 15) // 16

    def fire_chunk(ch, par, sem):
        c0 = t_start + ch * _K
        ntc = jnp.clip(t_end - c0, 0, _K)

        def fire(i, cc):
            k = i // 8
            g = i % 8
            pltpu.async_copy(
                table_hbm.at[pl.ds(g * 8, 8), pl.ds((c0 + k) * 128, 128)],
                stage_v.at[par, k, pl.ds(g * 8, 8)],
                sem,
            )
            return cc

        lax.fori_loop(0, ntc * 8, fire, 0)

    def drain_chunk(ch, par, sem):
        ntc = jnp.clip(t_end - (t_start + ch * _K), 0, _K)

        def drain(k, cc):
            pltpu.make_async_copy(
                table_hbm.at[pl.ds(0, D), pl.ds(0, 128)],
                stage_v.at[par, k], sem).wait()
            return cc

        lax.fori_loop(0, ntc, drain, 0)

    def do_chunk(ch, par):
        c0 = t_start + ch * _K
        ntc = jnp.clip(t_end - c0, 0, _K)

        def compact(j, off):
            mv = j * 16 + lane < nloc
            p = jnp.where(mv, pp_v[pl.ds(j * 16, 16)], 0)
            r = plsc.load_gather(idx_v, [p])
            t = lax.shift_right_logical(r, 7)
            m = (t >= c0) & (t < c0 + ntc) & mv
            plsc.store_compressed(pc_v.at[pl.ds(off, 16)], p, mask=m)
            return off + plsc.all_reduce_population_count(m)[0]

        n2 = lax.fori_loop(0, nloc_g, compact, 0)

        def serve(g, cc):
            mval = g * 16 + lane < n2
            pg = pc_v[pl.ds(g * 16, 16)]
            rg = plsc.load_gather(idx_v, [jnp.where(mval, pg, 0)])
            slot = jnp.where(mval, lax.shift_right_logical(rg, 7) - c0, 0)
            rmod = jnp.where(mval, rg & 127, 0)
            for c in range(D):
                cs = jnp.full((16,), c, jnp.int32)
                vals = plsc.load_gather(stage_v.at[par], [slot, cs, rmod])
                plsc.store_scatter(rows_v, [lane, cs], vals)
            pos_v[...] = jnp.where(mval, pg, _DUMP + lane)
            pltpu.async_copy(rows_v, out_hbm.at[pos_v], sem_o).wait()
            return cc

        lax.fori_loop(0, (n2 + 15) // 16, serve, 0)

    n_chunks = (t_end - t_start + _K - 1) // _K
    fire_chunk(0, 0, sem_a)

    def pair(i, carry):
        ch0 = 2 * i
        fire_chunk(ch0 + 1, 1, sem_b)
        drain_chunk(ch0, 0, sem_a)
        do_chunk(ch0, 0)
        fire_chunk(ch0 + 2, 0, sem_a)
        drain_chunk(ch0 + 1, 1, sem_b)
        do_chunk(ch0 + 1, 1)
        return carry

    lax.fori_loop(0, (n_chunks + 1) // 2, pair, 0)


def _mlp_t_body(h_ref, w1_ref, b1_ref, w2_ref, b2_ref, wo_ref, bo_ref,
                out_ref):
    h = h_ref[...][:, :D]
    z1 = lax.dot_general(w1_ref[...], h, (((1,), (1,)), ((), ())),
                         preferred_element_type=jnp.float32)
    z1 = z1 + b1_ref[...]
    h1 = 0.5 * z1 * (1.0 + lax.erf(z1 * 0.7071067811865476))
    z2 = jnp.dot(w2_ref[...], h1, preferred_element_type=jnp.float32)
    z2 = z2 + b2_ref[...]
    h2 = 0.5 * z2 * (1.0 + lax.erf(z2 * 0.7071067811865476))
    out = jnp.dot(wo_ref[...], h2, preferred_element_type=jnp.float32)
    out_ref[...] = out + bo_ref[...]


_BLK = 2048


def _mlp_t(h128, w1f, b1f, w2f, b2f, wout, bout):
    grid = (B // _BLK,)
    return pl.pallas_call(
        _mlp_t_body,
        grid=grid,
        in_specs=[
            pl.BlockSpec((_BLK, 128), lambda i: (i, 0)),
            pl.BlockSpec((H1, D), lambda i: (0, 0)),
            pl.BlockSpec((H1, 1), lambda i: (0, 0)),
            pl.BlockSpec((H2, H1), lambda i: (0, 0)),
            pl.BlockSpec((H2, 1), lambda i: (0, 0)),
            pl.BlockSpec((NC_OUT, H2), lambda i: (0, 0)),
            pl.BlockSpec((NC_OUT, 1), lambda i: (0, 0)),
        ],
        out_specs=pl.BlockSpec((NC_OUT, _BLK), lambda i: (0, i)),
        out_shape=jax.ShapeDtypeStruct((NC_OUT, B), jnp.float32),
    )(h128, w1f, b1f, w2f, b2f, wout, bout)


def kernel(x, emb, g0, be0, rm0, rv0, W1, b1, g1, be1, rm1, rv1,
           W2, b2, g2, be2, rm2, rv2, Wout, bout):
    s0 = g0 / jnp.sqrt(rv0 + EPS)
    t0 = be0 - rm0 * s0
    s1 = g1 / jnp.sqrt(rv1 + EPS)
    t1 = be1 - rm1 * s1
    s2 = g2 / jnp.sqrt(rv2 + EPS)
    t2 = be2 - rm2 * s2

    w1f = W1 * s0[None, :] * s1[:, None]
    b1f = (t0 @ W1.T + b1) * s1 + t1
    w2f = W2 * s2[:, None]
    b2f = b2 * s2 + t2

    idx = x[:, 0].astype(jnp.int32)
    rows = _sc_gather(emb.T, idx)
    out_t = _mlp_t(rows[:B], w1f, b1f[:, None], w2f, b2f[:, None],
                   Wout, bout[:, None])
    return out_t.T


# speedup vs baseline: 1.3998x; 1.3998x over previous
import functools

import jax
import jax.numpy as jnp
from jax import lax
from jax.experimental import pallas as pl
from jax.experimental.pallas import tpu as pltpu
from jax.experimental.pallas import tpu_sc as plsc

B = 16384
V = 1000000
D = 64
H1 = 384
H2 = 192
NC_OUT = 173
EPS = 1e-5

_NW = 32
_NT = (V + 127) // 128
_TPW = (_NT + _NW - 1) // _NW
_K = 4
_DUMP = B
_OUT_ROWS = B + 16


@functools.partial(
    pl.kernel,
    out_type=jax.ShapeDtypeStruct((_OUT_ROWS, 128), jnp.float32),
    mesh=plsc.VectorSubcoreMesh(core_axis_name="c", subcore_axis_name="s"),
    scratch_types=[
        pltpu.VMEM((B,), jnp.int32),
        pltpu.VMEM((B + 16,), jnp.int32),
        pltpu.VMEM((B + 16,), jnp.int32),
        pltpu.VMEM((2, _K, D, 128), jnp.float32),
        pltpu.VMEM((16, 128), jnp.float32),
        pltpu.VMEM((16,), jnp.int32),
        pltpu.SemaphoreType.DMA,
        pltpu.SemaphoreType.DMA,
        pltpu.SemaphoreType.DMA,
    ],
    compiler_params=pltpu.CompilerParams(
        use_tc_tiling_on_sc=True, needs_layout_passes=False,
        disable_bounds_checks=True),
)
def _sc_gather(table_hbm, idx_hbm, out_hbm, idx_v, pp_v, pc_v,
               stage_v, rows_v, pos_v, sem_a, sem_b, sem_o):
    wid = lax.axis_index("s") * 2 + lax.axis_index("c")
    t_start = wid * _TPW
    t_end = jnp.minimum(t_start + _TPW, _NT)
    pltpu.sync_copy(idx_hbm, idx_v)
    lane = lax.broadcasted_iota(jnp.int32, (16,), 0)

    def scan(i, off):
        r = idx_v[pl.ds(i * 16, 16)]
        t = lax.shift_right_logical(r, 7)
        m = (t >= t_start) & (t < t_end)
        p = lane + i * 16
        plsc.store_compressed(pp_v.at[pl.ds(off, 16)], p, mask=m)
        return off + plsc.all_reduce_population_count(m)[0]

    nloc = lax.fori_loop(0, B // 16, scan, 0)
    nloc_g = (nloc + 15) // 16

    def fire_chunk(ch, par, sem):
        c0 = t_start + ch * _K
        ntc = jnp.clip(t_end - c0, 0, _K)

        def fire(i, cc):
            k = i // 8
            g = i % 8
            pltpu.async_copy(
                table_hbm.at[pl.ds(g * 8, 8), pl.ds((c0 + k) * 128, 128)],
                stage_v.at[par, k, pl.ds(g * 8, 8)],
                sem,
            )
            return cc

        lax.fori_loop(0, ntc * 8, fire, 0)

    def drain_chunk(ch, par, sem):
        ntc = jnp.clip(t_end - (t_start + ch * _K), 0, _K)

        def drain(k, cc):
            pltpu.make_async_copy(
                table_hbm.at[pl.ds(0, D), pl.ds(0, 128)],
                stage_v.at[par, k], sem).wait()
            return cc

        lax.fori_loop(0, ntc, drain, 0)

    def do_chunk(ch, par):
        c0 = t_start + ch * _K
        ntc = jnp.clip(t_end - c0, 0, _K)

        def compact(j, off):
            mv = j * 16 + lane < nloc
            p = jnp.where(mv, pp_v[pl.ds(j * 16, 16)], 0)
            r = plsc.load_gather(idx_v, [p])
            t = lax.shift_right_logical(r, 7)
            m = (t >= c0) & (t < c0 + ntc) & mv
            plsc.store_compressed(pc_v.at[pl.ds(off, 16)], p, mask=m)
            return off + plsc.all_reduce_population_count(m)[0]

        n2 = lax.fori_loop(0, nloc_g, compact, 0)

        def serve(g, cc):
            mval = g * 16 + lane < n2
            pg = pc_v[pl.ds(g * 16, 16)]
            rg = plsc.load_gather(idx_v, [jnp.where(mval, pg, 0)])
            slot = jnp.where(mval, lax.shift_right_logical(rg, 7) - c0, 0)
            rmod = jnp.where(mval, rg & 127, 0)
            for c in range(D):
                cs = jnp.full((16,), c, jnp.int32)
                vals = plsc.load_gather(stage_v.at[par], [slot, cs, rmod])
                plsc.store_scatter(rows_v, [lane, cs], vals)
            pos_v[...] = jnp.where(mval, pg, _DUMP + lane)
            pltpu.async_copy(rows_v, out_hbm.at[pos_v], sem_o).wait()
            return cc

        lax.fori_loop(0, (n2 + 15) // 16, serve, 0)

    n_chunks = (t_end - t_start + _K - 1) // _K
    fire_chunk(0, 0, sem_a)

    def pair(i, carry):
        ch0 = 2 * i
        fire_chunk(ch0 + 1, 1, sem_b)
        drain_chunk(ch0, 0, sem_a)
        do_chunk(ch0, 0)
        fire_chunk(ch0 + 2, 0, sem_a)
        drain_chunk(ch0 + 1, 1, sem_b)
        do_chunk(ch0 + 1, 1)
        return carry

    lax.fori_loop(0, (n_chunks + 1) // 2, pair, 0)


def _mlp_t_body(h_ref, w1_ref, b1_ref, w2_ref, b2_ref, wo_ref, bo_ref,
                out_ref):
    h = h_ref[...][:, :D]
    z1 = lax.dot_general(w1_ref[...], h, (((1,), (1,)), ((), ())),
                         preferred_element_type=jnp.float32)
    z1 = z1 + b1_ref[...]
    h1 = 0.5 * z1 * (1.0 + lax.erf(z1 * 0.7071067811865476))
    z2 = jnp.dot(w2_ref[...], h1, preferred_element_type=jnp.float32)
    z2 = z2 + b2_ref[...]
    h2 = 0.5 * z2 * (1.0 + lax.erf(z2 * 0.7071067811865476))
    out = jnp.dot(wo_ref[...], h2, preferred_element_type=jnp.float32)
    out_ref[...] = out + bo_ref[...]


_BLK = 2048


def _mlp_t(h128, w1f, b1f, w2f, b2f, wout, bout):
    grid = (B // _BLK,)
    return pl.pallas_call(
        _mlp_t_body,
        grid=grid,
        in_specs=[
            pl.BlockSpec((_BLK, 128), lambda i: (i, 0)),
            pl.BlockSpec((H1, D), lambda i: (0, 0)),
            pl.BlockSpec((H1, 1), lambda i: (0, 0)),
            pl.BlockSpec((H2, H1), lambda i: (0, 0)),
            pl.BlockSpec((H2, 1), lambda i: (0, 0)),
            pl.BlockSpec((NC_OUT, H2), lambda i: (0, 0)),
            pl.BlockSpec((NC_OUT, 1), lambda i: (0, 0)),
        ],
        out_specs=pl.BlockSpec((NC_OUT, _BLK), lambda i: (0, i)),
        out_shape=jax.ShapeDtypeStruct((NC_OUT, B), jnp.float32),
    )(h128, w1f, b1f, w2f, b2f, wout, bout)


def kernel(x, emb, g0, be0, rm0, rv0, W1, b1, g1, be1, rm1, rv1,
           W2, b2, g2, be2, rm2, rv2, Wout, bout):
    s0 = g0 / jnp.sqrt(rv0 + EPS)
    t0 = be0 - rm0 * s0
    s1 = g1 / jnp.sqrt(rv1 + EPS)
    t1 = be1 - rm1 * s1
    s2 = g2 / jnp.sqrt(rv2 + EPS)
    t2 = be2 - rm2 * s2

    w1f = W1 * s0[None, :] * s1[:, None]
    b1f = (t0 @ W1.T + b1) * s1 + t1
    w2f = W2 * s2[:, None]
    b2f = b2 * s2 + t2

    idx = x[:, 0].astype(jnp.int32)
    rows = _sc_gather(emb.T, idx)
    out_t = _mlp_t(rows[:B], w1f, b1f[:, None], w2f, b2f[:, None],
                   Wout, bout[:, None])
    return out_t.T


# speedup vs baseline: 1.4086x; 1.0063x over previous
import functools

import jax
import jax.numpy as jnp
from jax import lax
from jax.experimental import pallas as pl
from jax.experimental.pallas import tpu as pltpu
from jax.experimental.pallas import tpu_sc as plsc

B = 16384
V = 1000000
D = 64
H1 = 384
H2 = 192
NC_OUT = 173
EPS = 1e-5

_NW = 32
_NT = (V + 127) // 128
_TPW = (_NT + _NW - 1) // _NW
_K = 4
_DUMP = B
_OUT_ROWS = B + 16


@functools.partial(
    pl.kernel,
    out_type=jax.ShapeDtypeStruct((_OUT_ROWS, 128), jnp.float32),
    mesh=plsc.VectorSubcoreMesh(core_axis_name="c", subcore_axis_name="s"),
    scratch_types=[
        pltpu.VMEM((B,), jnp.int32),
        pltpu.VMEM((B + 16,), jnp.int32),
        pltpu.VMEM((B + 16,), jnp.int32),
        pltpu.VMEM((2, _K, D, 128), jnp.float32),
        pltpu.VMEM((2, 16, 128), jnp.float32),
        pltpu.VMEM((2, 16), jnp.int32),
        pltpu.SemaphoreType.DMA,
        pltpu.SemaphoreType.DMA,
        pltpu.SemaphoreType.DMA,
    ],
    compiler_params=pltpu.CompilerParams(
        use_tc_tiling_on_sc=True, needs_layout_passes=False,
        disable_bounds_checks=True),
)
def _sc_gather(table_hbm, idx_hbm, out_hbm, idx_v, pp_v, pc_v,
               stage_v, rows_v, pos_v, sem_a, sem_b, sem_o):
    wid = lax.axis_index("s") * 2 + lax.axis_index("c")
    t_start = wid * _TPW
    t_end = jnp.minimum(t_start + _TPW, _NT)
    pltpu.sync_copy(idx_hbm, idx_v)
    lane = lax.broadcasted_iota(jnp.int32, (16,), 0)

    def fire_chunk(ch, par, sem):
        c0 = t_start + ch * _K
        ntc = jnp.clip(t_end - c0, 0, _K)

        def fire(i, cc):
            k = i // 8
            g = i % 8
            pltpu.async_copy(
                table_hbm.at[pl.ds(g * 8, 8), pl.ds((c0 + k) * 128, 128)],
                stage_v.at[par, k, pl.ds(g * 8, 8)],
                sem,
            )
            return cc

        lax.fori_loop(0, ntc * 8, fire, 0)

    def drain_chunk(ch, par, sem):
        ntc = jnp.clip(t_end - (t_start + ch * _K), 0, _K)

        def drain(k, cc):
            pltpu.make_async_copy(
                table_hbm.at[pl.ds(0, D), pl.ds(0, 128)],
                stage_v.at[par, k], sem).wait()
            return cc

        lax.fori_loop(0, ntc, drain, 0)

    def do_chunk(ch, par):
        c0 = t_start + ch * _K
        ntc = jnp.clip(t_end - c0, 0, _K)

        def compact(j, off):
            mv = j * 16 + lane < nloc
            p = jnp.where(mv, pp_v[pl.ds(j * 16, 16)], 0)
            r = plsc.load_gather(idx_v, [p])
            t = lax.shift_right_logical(r, 7)
            m = (t >= c0) & (t < c0 + ntc) & mv
            plsc.store_compressed(pc_v.at[pl.ds(off, 16)], p, mask=m)
            return off + plsc.all_reduce_population_count(m)[0]

        n2 = lax.fori_loop(0, nloc_g, compact, 0)

        def serve(g, cc):
            sbuf = g & 1
            def wait_prev(q, qq):
                pltpu.make_async_copy(
                    rows_v.at[0], out_hbm.at[pl.ds(0, 16)], sem_o).wait()
                return qq

            lax.fori_loop(0, jnp.where(g >= 2, 1, 0), wait_prev, 0)
            mval = g * 16 + lane < n2
            pg = pc_v[pl.ds(g * 16, 16)]
            rg = plsc.load_gather(idx_v, [jnp.where(mval, pg, 0)])
            slot = jnp.where(mval, lax.shift_right_logical(rg, 7) - c0, 0)
            rmod = jnp.where(mval, rg & 127, 0)
            for c in range(D):
                cs = jnp.full((16,), c, jnp.int32)
                vals = plsc.load_gather(stage_v.at[par], [slot, cs, rmod])
                plsc.store_scatter(rows_v.at[sbuf], [lane, cs], vals)
            pos_v[sbuf, :] = jnp.where(mval, pg, _DUMP + lane)
            pltpu.async_copy(rows_v.at[sbuf], out_hbm.at[pos_v.at[sbuf]],
                             sem_o)
            return cc

        ng = (n2 + 15) // 16
        lax.fori_loop(0, ng, serve, 0)

        def drain_rows(q, qq):
            pltpu.make_async_copy(
                rows_v.at[0], out_hbm.at[pl.ds(0, 16)], sem_o).wait()
            return qq

        lax.fori_loop(0, jnp.minimum(ng, 2), drain_rows, 0)

    n_chunks = (t_end - t_start + _K - 1) // _K
    fire_chunk(0, 0, sem_a)

    def scan(i, off):
        r = idx_v[pl.ds(i * 16, 16)]
        t = lax.shift_right_logical(r, 7)
        m = (t >= t_start) & (t < t_end)
        p = lane + i * 16
        plsc.store_compressed(pp_v.at[pl.ds(off, 16)], p, mask=m)
        return off + plsc.all_reduce_population_count(m)[0]

    nloc = lax.fori_loop(0, B // 16, scan, 0)
    nloc_g = (nloc + 15) // 16


    def pair(i, carry):
        ch0 = 2 * i
        fire_chunk(ch0 + 1, 1, sem_b)
        drain_chunk(ch0, 0, sem_a)
        do_chunk(ch0, 0)
        fire_chunk(ch0 + 2, 0, sem_a)
        drain_chunk(ch0 + 1, 1, sem_b)
        do_chunk(ch0 + 1, 1)
        return carry

    lax.fori_loop(0, (n_chunks + 1) // 2, pair, 0)


def _mlp_t_body(h_ref, w1_ref, b1_ref, w2_ref, b2_ref, wo_ref, bo_ref,
                out_ref):
    h = h_ref[...][:, :D]
    z1 = lax.dot_general(w1_ref[...], h, (((1,), (1,)), ((), ())),
                         preferred_element_type=jnp.float32)
    z1 = z1 + b1_ref[...]
    h1 = 0.5 * z1 * (1.0 + lax.erf(z1 * 0.7071067811865476))
    z2 = jnp.dot(w2_ref[...], h1, preferred_element_type=jnp.float32)
    z2 = z2 + b2_ref[...]
    h2 = 0.5 * z2 * (1.0 + lax.erf(z2 * 0.7071067811865476))
    out = jnp.dot(wo_ref[...], h2, preferred_element_type=jnp.float32)
    out_ref[...] = out + bo_ref[...]


_BLK = 2048


def _mlp_t(h128, w1f, b1f, w2f, b2f, wout, bout):
    grid = (B // _BLK,)
    return pl.pallas_call(
        _mlp_t_body,
        grid=grid,
        in_specs=[
            pl.BlockSpec((_BLK, 128), lambda i: (i, 0)),
            pl.BlockSpec((H1, D), lambda i: (0, 0)),
            pl.BlockSpec((H1, 1), lambda i: (0, 0)),
            pl.BlockSpec((H2, H1), lambda i: (0, 0)),
            pl.BlockSpec((H2, 1), lambda i: (0, 0)),
            pl.BlockSpec((NC_OUT, H2), lambda i: (0, 0)),
            pl.BlockSpec((NC_OUT, 1), lambda i: (0, 0)),
        ],
        out_specs=pl.BlockSpec((NC_OUT, _BLK), lambda i: (0, i)),
        out_shape=jax.ShapeDtypeStruct((NC_OUT, B), jnp.float32),
    )(h128, w1f, b1f, w2f, b2f, wout, bout)


def kernel(x, emb, g0, be0, rm0, rv0, W1, b1, g1, be1, rm1, rv1,
           W2, b2, g2, be2, rm2, rv2, Wout, bout):
    s0 = g0 / jnp.sqrt(rv0 + EPS)
    t0 = be0 - rm0 * s0
    s1 = g1 / jnp.sqrt(rv1 + EPS)
    t1 = be1 - rm1 * s1
    s2 = g2 / jnp.sqrt(rv2 + EPS)
    t2 = be2 - rm2 * s2

    w1f = W1 * s0[None, :] * s1[:, None]
    b1f = (t0 @ W1.T + b1) * s1 + t1
    w2f = W2 * s2[:, None]
    b2f = b2 * s2 + t2

    idx = x[:, 0].astype(jnp.int32)
    rows = _sc_gather(emb.T, idx)
    out_t = _mlp_t(rows[:B], w1f, b1f[:, None], w2f, b2f[:, None],
                   Wout, bout[:, None])
    return out_t.T
